# Initial kernel scaffold; baseline (speedup 1.0000x reference)
import jax
import jax.numpy as jnp
from jax.experimental import pallas as pl


def kernel(in_feat, edge_index, W0, b0, W1, b1, W2, b2, noise):
    raise NotImplementedError("write your pallas kernel here")


# speedup vs baseline: 5.9272x; 5.9272x over previous
import functools

import jax
import jax.numpy as jnp
from jax import lax
from jax.experimental import pallas as pl
from jax.experimental.pallas import tpu as pltpu
from jax.experimental.pallas import tpu_sc as plsc

_CH = 128


def _sc_info():
    info = plsc.get_sparse_core_info()
    return info.num_cores, info.num_subcores


def _zero_fill(ref, nrows, ncols):
    per_row = ncols // 16

    def body(i, _):
        r = i // per_row
        cpos = i % per_row
        ref[r, pl.ds(cpos * 16, 16)] = jnp.zeros((16,), jnp.float32)
        return 0

    lax.fori_loop(0, nrows * per_row, body, 0)


def _fill_flat(ref, nelems, value):
    def body(i, _):
        ref[pl.ds(i * 16, 16)] = jnp.full((16,), value, jnp.float32)
        return 0

    lax.fori_loop(0, nelems // 16, body, 0)


def _sc_degrees(src, dst, n):
    e = src.shape[0]
    nchunk = e // _CH
    nc, ns = _sc_info()
    nw = nc * ns
    nloop = pl.cdiv(nchunk, nw)
    per_tile = ((n + ns - 1) // ns + 7) // 8 * 8
    npad = per_tile * ns
    mesh = plsc.VectorSubcoreMesh(core_axis_name="c", subcore_axis_name="s")

    @functools.partial(
        pl.kernel,
        out_type=(
            jax.ShapeDtypeStruct((nc * npad,), jnp.float32),
            jax.ShapeDtypeStruct((nc * npad,), jnp.float32),
        ),
        mesh=mesh,
        scratch_types=[
            pltpu.VMEM((_CH,), jnp.int32),
            pltpu.VMEM((_CH,), jnp.int32),
            pltpu.VMEM((_CH,), jnp.float32),
            pltpu.VMEM((per_tile,), jnp.float32),
            pltpu.VMEM_SHARED((npad,), jnp.float32),
            pltpu.VMEM_SHARED((npad,), jnp.float32),
        ],
    )
    def deg_kernel(src_hbm, dst_hbm, out_s, out_d, sidx, didx, ones, buf,
                   deg_s, deg_d):
        c = lax.axis_index("c")
        s = lax.axis_index("s")
        wid = s * nc + c
        base = s * per_tile

        _fill_flat(ones, _CH, 1.0)
        _fill_flat(buf, per_tile, 0.0)
        pltpu.sync_copy(buf, deg_s.at[pl.ds(base, per_tile)])
        pltpu.sync_copy(buf, deg_d.at[pl.ds(base, per_tile)])
        plsc.subcore_barrier()

        def body(t, _):
            cc = wid + t * nw

            @pl.when(cc < nchunk)
            def _():
                off = pl.multiple_of(cc * _CH, _CH)
                pltpu.sync_copy(src_hbm.at[pl.ds(off, _CH)], sidx)
                pltpu.sync_copy(ones, deg_s.at[sidx], add=True)
                pltpu.sync_copy(dst_hbm.at[pl.ds(off, _CH)], didx)
                pltpu.sync_copy(ones, deg_d.at[didx], add=True)

            return 0

        lax.fori_loop(0, nloop, body, 0)
        plsc.subcore_barrier()

        obase = c * npad + base
        pltpu.sync_copy(deg_s.at[pl.ds(base, per_tile)], buf)
        pltpu.sync_copy(buf, out_s.at[pl.ds(obase, per_tile)])
        pltpu.sync_copy(deg_d.at[pl.ds(base, per_tile)], buf)
        pltpu.sync_copy(buf, out_d.at[pl.ds(obase, per_tile)])

    out_s, out_d = deg_kernel(src, dst)
    return out_s.reshape(nc, npad), out_d.reshape(nc, npad)


def _sc_scatter(src, dst, table, n):
    e = src.shape[0]
    h = table.shape[1]
    nchunk = e // _CH
    nc, ns = _sc_info()
    nw = nc * ns
    nloop = pl.cdiv(nchunk, nw)
    per_tile = ((n + ns - 1) // ns + 7) // 8 * 8
    npad = per_tile * ns
    full = per_tile // _CH
    tail = per_tile - full * _CH
    mesh = plsc.VectorSubcoreMesh(core_axis_name="c", subcore_axis_name="s")

    @functools.partial(
        pl.kernel,
        out_type=jax.ShapeDtypeStruct((nc, npad, h), jnp.float32),
        mesh=mesh,
        scratch_types=[
            pltpu.VMEM((_CH,), jnp.int32),
            pltpu.VMEM((_CH,), jnp.int32),
            pltpu.VMEM((_CH, h), jnp.float32),
            pltpu.VMEM_SHARED((npad, h), jnp.float32),
            pltpu.SemaphoreType.DMA,
        ],
        compiler_params=pltpu.CompilerParams(use_tc_tiling_on_sc=False),
    )
    def scatter_kernel(src_hbm, dst_hbm, table_hbm, out_hbm, sidx, didx, rows,
                       agg, sem):
        c = lax.axis_index("c")
        s = lax.axis_index("s")
        wid = s * nc + c
        base = s * per_tile

        _zero_fill(rows, _CH, h)
        for k in range(full):
            pltpu.sync_copy(rows, agg.at[pl.ds(base + k * _CH, _CH)])
        if tail:
            pltpu.sync_copy(rows.at[pl.ds(0, tail)],
                            agg.at[pl.ds(base + full * _CH, tail)])
        plsc.subcore_barrier()

        def body(t, _):
            cc = wid + t * nw

            @pl.when(cc < nchunk)
            def _():
                off = pl.multiple_of(cc * _CH, _CH)
                pltpu.sync_copy(src_hbm.at[pl.ds(off, _CH)], sidx)
                pltpu.async_copy(table_hbm.at[sidx], rows, sem).wait()
                pltpu.sync_copy(dst_hbm.at[pl.ds(off, _CH)], didx)
                pltpu.sync_copy(rows, agg.at[didx], add=True)

            return 0

        lax.fori_loop(0, nloop, body, 0)
        plsc.subcore_barrier()

        for k in range(full + 1):
            sz = _CH if k < full else tail
            if sz:
                ro = base + k * _CH
                pltpu.sync_copy(agg.at[pl.ds(ro, sz)], rows.at[pl.ds(0, sz)])
                pltpu.sync_copy(rows.at[pl.ds(0, sz)],
                                out_hbm.at[c, pl.ds(ro, sz)])

    return scatter_kernel(src, dst, table)


def _norm_from(deg_ref):
    d = jnp.sum(deg_ref[...], axis=1, keepdims=True)
    return lax.rsqrt(jnp.maximum(d, 1.0))


def _tc_table0(x, w0, deg_s, n, blk):
    d_in = x.shape[1]
    h = w0.shape[1]

    def body(x_ref, w_ref, ds_ref, o_ref):
        sn = _norm_from(ds_ref)
        o_ref[...] = (
            jnp.dot(x_ref[...], w_ref[...], preferred_element_type=jnp.float32)
            * sn)

    return pl.pallas_call(
        body,
        grid=(n // blk,),
        in_specs=[
            pl.BlockSpec((blk, d_in), lambda i: (i, 0)),
            pl.BlockSpec((d_in, h), lambda i: (0, 0)),
            pl.BlockSpec((blk, 2), lambda i: (i, 0)),
        ],
        out_specs=pl.BlockSpec((blk, h), lambda i: (i, 0)),
        out_shape=jax.ShapeDtypeStruct((n, h), jnp.float32),
    )(x, w0, deg_s)


def _tc_table1(agg_p, deg_s, deg_d, b0, n, blk):
    h = agg_p.shape[2]

    def body(p_ref, ds_ref, dd_ref, b_ref, o_ref):
        a = p_ref[0] + p_ref[1]
        hid = jnp.maximum(a * _norm_from(dd_ref) + b_ref[...], 0.0)
        o_ref[...] = hid * _norm_from(ds_ref)

    return pl.pallas_call(
        body,
        grid=(n // blk,),
        in_specs=[
            pl.BlockSpec((2, blk, h), lambda i: (0, i, 0)),
            pl.BlockSpec((blk, 2), lambda i: (i, 0)),
            pl.BlockSpec((blk, 2), lambda i: (i, 0)),
            pl.BlockSpec((1, h), lambda i: (0, 0)),
        ],
        out_specs=pl.BlockSpec((blk, h), lambda i: (i, 0)),
        out_shape=jax.ShapeDtypeStruct((n, h), jnp.float32),
    )(agg_p, deg_s, deg_d, b0)


def _tc_final(agg_p, deg_d, w1, b1, w2, b2, noise, n, blk):
    h = agg_p.shape[2]
    ho = w1.shape[1]

    def body(p_ref, dd_ref, w1_ref, b1_ref, w2_ref, b2_ref, nz_ref, o_ref):
        m = (p_ref[0] + p_ref[1]) * _norm_from(dd_ref)
        mean = jnp.dot(m, w1_ref[...],
                       preferred_element_type=jnp.float32) + b1_ref[...]
        log_std = jnp.dot(m, w2_ref[...],
                          preferred_element_type=jnp.float32) + b2_ref[...]
        o_ref[...] = mean + nz_ref[...] * jnp.exp(log_std)

    return pl.pallas_call(
        body,
        grid=(n // blk,),
        in_specs=[
            pl.BlockSpec((2, blk, h), lambda i: (0, i, 0)),
            pl.BlockSpec((blk, 2), lambda i: (i, 0)),
            pl.BlockSpec((h, ho), lambda i: (0, 0)),
            pl.BlockSpec((1, ho), lambda i: (0, 0)),
            pl.BlockSpec((h, ho), lambda i: (0, 0)),
            pl.BlockSpec((1, ho), lambda i: (0, 0)),
            pl.BlockSpec((blk, ho), lambda i: (i, 0)),
        ],
        out_specs=pl.BlockSpec((blk, ho), lambda i: (i, 0)),
        out_shape=jax.ShapeDtypeStruct((n, ho), jnp.float32),
    )(agg_p, deg_d, w1, b1, w2, b2, noise)


def kernel(in_feat, edge_index, W0, b0, W1, b1, W2, b2, noise):
    n = in_feat.shape[0]
    blk = 1000
    src = edge_index[0].astype(jnp.int32)
    dst = edge_index[1].astype(jnp.int32)

    deg_s_p, deg_d_p = _sc_degrees(src, dst, n)
    deg_s = deg_s_p.T
    deg_d = deg_d_p.T

    table0 = _tc_table0(in_feat, W0, deg_s, n, blk)
    agg0_p = _sc_scatter(src, dst, table0, n)
    table1 = _tc_table1(agg0_p, deg_s, deg_d, b0[None, :], n, blk)
    agg1_p = _sc_scatter(src, dst, table1, n)
    return _tc_final(agg1_p, deg_d, W1, b1[None, :], W2, b2[None, :], noise,
                     n, blk)
